# baseline (speedup 1.0000x reference)
import functools

import jax
import jax.numpy as jnp
from jax import lax
from jax.experimental import pallas as pl
from jax.experimental.pallas import tpu as pltpu
from jax.experimental.pallas import tpu_sc as plsc

B = 4
T = 8192
D = 768
ROWS = B * T
NW = 32
RPW = ROWS // NW
W = 64
NCHUNK = RPW // W
WPB = T // RPW


def _emb_kernel(tok_hbm, idx_hbm, pos_hbm, out_hbm, idx_v, rows_v, pos_v, sem):
    wid = lax.axis_index("s") * 2 + lax.axis_index("c")
    base = wid * RPW
    t0 = (wid % WPB) * RPW
    pltpu.sync_copy(idx_hbm.at[pl.ds(base, RPW)], idx_v)

    @pl.loop(0, NCHUNK)
    def _chunk(c):
        r0 = pl.multiple_of(c * W, W)
        pltpu.async_copy(tok_hbm.at[idx_v.at[pl.ds(r0, W)]], rows_v, sem).wait()
        pltpu.sync_copy(pos_hbm.at[pl.ds(t0 + r0, W)], pos_v)

        @pl.loop(0, W)
        def _row(r):
            row = rows_v.at[r]
            prow = pos_v.at[r]
            for d in range(0, D, 16):
                sl = pl.ds(d, 16)
                row[sl] = row[sl] + prow[sl]

        pltpu.sync_copy(rows_v, out_hbm.at[pl.ds(base + r0, W)])


@jax.jit
def kernel(x, token_emb, pos_emb):
    xf = x.reshape(ROWS).astype(jnp.int32)
    mesh = plsc.VectorSubcoreMesh(core_axis_name="c", subcore_axis_name="s")
    run = functools.partial(
        pl.kernel,
        out_type=jax.ShapeDtypeStruct((ROWS, D), jnp.float32),
        mesh=mesh,
        scratch_types=[
            pltpu.VMEM((RPW,), jnp.int32),
            pltpu.VMEM((W, D), jnp.float32),
            pltpu.VMEM((W, D), jnp.float32),
            pltpu.SemaphoreType.DMA,
        ],
    )(_emb_kernel)
    out = run(token_emb, xf, pos_emb)
    return out.reshape(B, T, D)


# speedup vs baseline: 1.7350x; 1.7350x over previous
import functools

import jax
import jax.numpy as jnp
from jax import lax
from jax.experimental import pallas as pl
from jax.experimental.pallas import tpu as pltpu
from jax.experimental.pallas import tpu_sc as plsc

B = 4
T = 8192
D = 768
ROWS = B * T
NW = 32
TPW = T // NW
W = 16
NCHUNK = TPW // W


def _issue_gathers(tok_hbm, idx_v, rows_v, buf, c, gsem):
    for b in range(B):
        pltpu.async_copy(
            tok_hbm.at[idx_v.at[pl.ds(b * TPW + c * W, W)]],
            rows_v.at[buf, pl.ds(b * W, W)],
            gsem,
        )


def _wait_gathers(tok_hbm, idx_v, rows_v, buf, c, gsem):
    for b in range(B):
        pltpu.make_async_copy(
            tok_hbm.at[idx_v.at[pl.ds(b * TPW + c * W, W)]],
            rows_v.at[buf, pl.ds(b * W, W)],
            gsem,
        ).wait()


def _issue_pos(pos_hbm, pos_v, buf, t0, c, psem):
    pltpu.async_copy(pos_hbm.at[pl.ds(t0 + c * W, W)], pos_v.at[buf], psem)


def _wait_pos(pos_hbm, pos_v, buf, t0, c, psem):
    pltpu.make_async_copy(
        pos_hbm.at[pl.ds(t0 + c * W, W)], pos_v.at[buf], psem
    ).wait()


def _issue_stores(out_hbm, rows_v, buf, t0, c, ssem):
    for b in range(B):
        pltpu.async_copy(
            rows_v.at[buf, pl.ds(b * W, W)],
            out_hbm.at[pl.ds(b * T + t0 + c * W, W)],
            ssem,
        )


def _wait_stores(out_hbm, rows_v, buf, t0, c, ssem):
    for b in range(B):
        pltpu.make_async_copy(
            rows_v.at[buf, pl.ds(b * W, W)],
            out_hbm.at[pl.ds(b * T + t0 + c * W, W)],
            ssem,
        ).wait()


def _compute(rows_v, pos_v, buf):
    @pl.loop(0, W)
    def _row(r):
        for d in range(0, D, 16):
            sl = pl.ds(d, 16)
            pv = pos_v[buf, r, sl]
            for b in range(B):
                plsc.addupdate(rows_v.at[buf, b * W + r, sl], pv)


def _emb_kernel(tok_hbm, idx_hbm, pos_hbm, out_hbm, idx_v, rows_v, pos_v,
                gsem0, gsem1, psem0, psem1, ssem0, ssem1):
    wid = lax.axis_index("s") * 2 + lax.axis_index("c")
    t0 = wid * TPW
    gsem = (gsem0, gsem1)
    psem = (psem0, psem1)
    ssem = (ssem0, ssem1)

    for b in range(B):
        pltpu.sync_copy(idx_hbm.at[pl.ds(b * T + t0, TPW)],
                        idx_v.at[pl.ds(b * TPW, TPW)])

    _issue_gathers(tok_hbm, idx_v, rows_v, 0, 0, gsem[0])
    _issue_pos(pos_hbm, pos_v, 0, t0, 0, psem[0])

    _issue_gathers(tok_hbm, idx_v, rows_v, 1, 1, gsem[1])
    _issue_pos(pos_hbm, pos_v, 1, t0, 1, psem[1])
    _wait_gathers(tok_hbm, idx_v, rows_v, 0, 0, gsem[0])
    _wait_pos(pos_hbm, pos_v, 0, t0, 0, psem[0])
    _compute(rows_v, pos_v, 0)
    _issue_stores(out_hbm, rows_v, 0, t0, 0, ssem[0])

    @pl.loop(1, NCHUNK - 1, step=2)
    def _chunks(c0):
        for cp in range(2):
            c = c0 + cp
            P = (1 + cp) % 2
            Q = 1 - P
            _wait_stores(out_hbm, rows_v, Q, t0, c - 1, ssem[Q])
            _issue_gathers(tok_hbm, idx_v, rows_v, Q, c + 1, gsem[Q])
            _issue_pos(pos_hbm, pos_v, Q, t0, c + 1, psem[Q])
            _wait_gathers(tok_hbm, idx_v, rows_v, P, c, gsem[P])
            _wait_pos(pos_hbm, pos_v, P, t0, c, psem[P])
            _compute(rows_v, pos_v, P)
            _issue_stores(out_hbm, rows_v, P, t0, c, ssem[P])

    last = NCHUNK - 1
    _wait_stores(out_hbm, rows_v, 0, t0, last - 1, ssem[0])
    _wait_gathers(tok_hbm, idx_v, rows_v, 1, last, gsem[1])
    _wait_pos(pos_hbm, pos_v, 1, t0, last, psem[1])
    _compute(rows_v, pos_v, 1)
    _issue_stores(out_hbm, rows_v, 1, t0, last, ssem[1])
    _wait_stores(out_hbm, rows_v, 1, t0, last, ssem[1])


@jax.jit
def kernel(x, token_emb, pos_emb):
    xf = x.reshape(ROWS).astype(jnp.int32)
    mesh = plsc.VectorSubcoreMesh(core_axis_name="c", subcore_axis_name="s")
    run = functools.partial(
        pl.kernel,
        out_type=jax.ShapeDtypeStruct((ROWS, D), jnp.float32),
        mesh=mesh,
        scratch_types=[
            pltpu.VMEM((B * TPW,), jnp.int32),
            pltpu.VMEM((2, B * W, D), jnp.float32),
            pltpu.VMEM((2, W, D), jnp.float32),
            pltpu.SemaphoreType.DMA,
            pltpu.SemaphoreType.DMA,
            pltpu.SemaphoreType.DMA,
            pltpu.SemaphoreType.DMA,
            pltpu.SemaphoreType.DMA,
            pltpu.SemaphoreType.DMA,
        ],
    )(_emb_kernel)
    out = run(token_emb, xf, pos_emb)
    return out.reshape(B, T, D)


# speedup vs baseline: 1.7377x; 1.0015x over previous
import functools

import jax
import jax.numpy as jnp
from jax import lax
from jax.experimental import pallas as pl
from jax.experimental.pallas import tpu as pltpu
from jax.experimental.pallas import tpu_sc as plsc

B = 4
T = 8192
D = 768
ROWS = B * T
NW = 32
TPW = T // NW
W = 16
NCHUNK = TPW // W


def _issue_gathers(tok_hbm, idx_v, rows_v, buf, c, gsem):
    pltpu.async_copy(
        tok_hbm.at[idx_v.at[pl.ds(c * (B * W), B * W)]],
        rows_v.at[buf],
        gsem,
    )


def _wait_gathers(tok_hbm, idx_v, rows_v, buf, c, gsem):
    pltpu.make_async_copy(
        tok_hbm.at[idx_v.at[pl.ds(c * (B * W), B * W)]],
        rows_v.at[buf],
        gsem,
    ).wait()


def _issue_pos(pos_hbm, pos_v, buf, t0, c, psem):
    pltpu.async_copy(pos_hbm.at[pl.ds(t0 + c * W, W)], pos_v.at[buf], psem)


def _wait_pos(pos_hbm, pos_v, buf, t0, c, psem):
    pltpu.make_async_copy(
        pos_hbm.at[pl.ds(t0 + c * W, W)], pos_v.at[buf], psem
    ).wait()


def _issue_stores(out_hbm, rows_v, buf, t0, c, ssem):
    for b in range(B):
        pltpu.async_copy(
            rows_v.at[buf, pl.ds(b * W, W)],
            out_hbm.at[pl.ds(b * T + t0 + c * W, W)],
            ssem,
        )


def _wait_stores(out_hbm, rows_v, buf, t0, c, ssem):
    for b in range(B):
        pltpu.make_async_copy(
            rows_v.at[buf, pl.ds(b * W, W)],
            out_hbm.at[pl.ds(b * T + t0 + c * W, W)],
            ssem,
        ).wait()


def _compute(rows_v, pos_v, buf):
    @pl.loop(0, W)
    def _row(r):
        for d in range(0, D, 16):
            sl = pl.ds(d, 16)
            pv = pos_v[buf, r, sl]
            for b in range(B):
                plsc.addupdate(rows_v.at[buf, b * W + r, sl], pv)


def _emb_kernel(tok_hbm, idx_hbm, pos_hbm, out_hbm, idx_t, idx_v, rows_v,
                pos_v, gsem0, gsem1, psem0, psem1, ssem0, ssem1):
    wid = lax.axis_index("s") * 2 + lax.axis_index("c")
    t0 = wid * TPW
    gsem = (gsem0, gsem1)
    psem = (psem0, psem1)
    ssem = (ssem0, ssem1)

    for b in range(B):
        pltpu.sync_copy(idx_hbm.at[pl.ds(b * T + t0, TPW)],
                        idx_t.at[pl.ds(b * TPW, TPW)])
    for c in range(NCHUNK):
        for b in range(B):
            for r in range(0, W, 16):
                idx_v[pl.ds(c * B * W + b * W + r, 16)] = (
                    idx_t[pl.ds(b * TPW + c * W + r, 16)])

    _issue_gathers(tok_hbm, idx_v, rows_v, 0, 0, gsem[0])
    _issue_pos(pos_hbm, pos_v, 0, t0, 0, psem[0])

    _issue_gathers(tok_hbm, idx_v, rows_v, 1, 1, gsem[1])
    _issue_pos(pos_hbm, pos_v, 1, t0, 1, psem[1])
    _wait_gathers(tok_hbm, idx_v, rows_v, 0, 0, gsem[0])
    _wait_pos(pos_hbm, pos_v, 0, t0, 0, psem[0])
    _compute(rows_v, pos_v, 0)
    _issue_stores(out_hbm, rows_v, 0, t0, 0, ssem[0])

    @pl.loop(1, NCHUNK - 1, step=2)
    def _chunks(c0):
        for cp in range(2):
            c = c0 + cp
            P = (1 + cp) % 2
            Q = 1 - P
            _wait_stores(out_hbm, rows_v, Q, t0, c - 1, ssem[Q])
            _issue_gathers(tok_hbm, idx_v, rows_v, Q, c + 1, gsem[Q])
            _issue_pos(pos_hbm, pos_v, Q, t0, c + 1, psem[Q])
            _wait_gathers(tok_hbm, idx_v, rows_v, P, c, gsem[P])
            _wait_pos(pos_hbm, pos_v, P, t0, c, psem[P])
            _compute(rows_v, pos_v, P)
            _issue_stores(out_hbm, rows_v, P, t0, c, ssem[P])

    last = NCHUNK - 1
    _wait_stores(out_hbm, rows_v, 0, t0, last - 1, ssem[0])
    _wait_gathers(tok_hbm, idx_v, rows_v, 1, last, gsem[1])
    _wait_pos(pos_hbm, pos_v, 1, t0, last, psem[1])
    _compute(rows_v, pos_v, 1)
    _issue_stores(out_hbm, rows_v, 1, t0, last, ssem[1])
    _wait_stores(out_hbm, rows_v, 1, t0, last, ssem[1])


@jax.jit
def kernel(x, token_emb, pos_emb):
    xf = x.reshape(ROWS).astype(jnp.int32)
    mesh = plsc.VectorSubcoreMesh(core_axis_name="c", subcore_axis_name="s")
    run = functools.partial(
        pl.kernel,
        out_type=jax.ShapeDtypeStruct((ROWS, D), jnp.float32),
        mesh=mesh,
        scratch_types=[
            pltpu.VMEM((B * TPW,), jnp.int32),
            pltpu.VMEM((B * TPW,), jnp.int32),
            pltpu.VMEM((2, B * W, D), jnp.float32),
            pltpu.VMEM((2, W, D), jnp.float32),
            pltpu.SemaphoreType.DMA,
            pltpu.SemaphoreType.DMA,
            pltpu.SemaphoreType.DMA,
            pltpu.SemaphoreType.DMA,
            pltpu.SemaphoreType.DMA,
            pltpu.SemaphoreType.DMA,
        ],
    )(_emb_kernel)
    out = run(token_emb, xf, pos_emb)
    return out.reshape(B, T, D)


# speedup vs baseline: 1.8295x; 1.0528x over previous
import functools

import jax
import jax.numpy as jnp
from jax import lax
from jax.experimental import pallas as pl
from jax.experimental.pallas import tpu as pltpu
from jax.experimental.pallas import tpu_sc as plsc

B = 4
T = 8192
D = 768
ROWS = B * T
NW = 32
TPW = T // NW
W = 8
CR = B * W
NCHUNK = TPW // W
NBUF = 3


def _issue_gather(tok_hbm, idx_v, rows_v, buf, c, gsems):
    pltpu.async_copy(
        tok_hbm.at[idx_v.at[pl.ds(c * CR, CR)]],
        rows_v.at[buf],
        gsems[buf],
    )


def _wait_gather(tok_hbm, idx_v, rows_v, buf, c, gsems):
    pltpu.make_async_copy(
        tok_hbm.at[idx_v.at[pl.ds(c * CR, CR)]],
        rows_v.at[buf],
        gsems[buf],
    ).wait()


def _issue_pos(pos_hbm, pos_v, buf, t0, c, psems):
    pltpu.async_copy(pos_hbm.at[pl.ds(t0 + c * W, W)], pos_v.at[buf],
                     psems[buf])


def _wait_pos(pos_hbm, pos_v, buf, t0, c, psems):
    pltpu.make_async_copy(
        pos_hbm.at[pl.ds(t0 + c * W, W)], pos_v.at[buf], psems[buf]
    ).wait()


def _issue_stores(out_hbm, rows_v, buf, t0, c, ssems):
    for b in range(B):
        pltpu.async_copy(
            rows_v.at[buf, pl.ds(b * W, W)],
            out_hbm.at[pl.ds(b * T + t0 + c * W, W)],
            ssems[buf],
        )


def _wait_stores(out_hbm, rows_v, buf, t0, c, ssems):
    for b in range(B):
        pltpu.make_async_copy(
            rows_v.at[buf, pl.ds(b * W, W)],
            out_hbm.at[pl.ds(b * T + t0 + c * W, W)],
            ssems[buf],
        ).wait()


def _compute(rows_v, pos_v, buf):
    @pl.loop(0, W)
    def _row(r):
        for d in range(0, D, 16):
            sl = pl.ds(d, 16)
            pv = pos_v[buf, r, sl]
            for b in range(B):
                plsc.addupdate(rows_v.at[buf, b * W + r, sl], pv)


def _emb_kernel(tok_hbm, idx_hbm, pos_hbm, out_hbm, idx_v, rows_v, pos_v,
                gsem0, gsem1, gsem2, psem0, psem1, psem2,
                ssem0, ssem1, ssem2):
    wid = lax.axis_index("s") * 2 + lax.axis_index("c")
    t0 = wid * TPW
    gsems = (gsem0, gsem1, gsem2)
    psems = (psem0, psem1, psem2)
    ssems = (ssem0, ssem1, ssem2)

    pltpu.sync_copy(idx_hbm.at[pl.ds(wid * B * TPW, B * TPW)], idx_v)

    def head(c):
        nb = (c + 2) % NBUF
        if c >= 1:
            _wait_stores(out_hbm, rows_v, nb, t0, c - 1, ssems)
        _issue_gather(tok_hbm, idx_v, rows_v, nb, c + 2, gsems)
        _issue_pos(pos_hbm, pos_v, nb, t0, c + 2, psems)

    def tail(c):
        P = c % NBUF
        _wait_gather(tok_hbm, idx_v, rows_v, P, c, gsems)
        _wait_pos(pos_hbm, pos_v, P, t0, c, psems)
        _compute(rows_v, pos_v, P)
        _issue_stores(out_hbm, rows_v, P, t0, c, ssems)

    for c in range(2):
        _issue_gather(tok_hbm, idx_v, rows_v, c, c, gsems)
        _issue_pos(pos_hbm, pos_v, c, t0, c, psems)
    for c in range(NBUF):
        head(c)
        tail(c)

    @pl.loop(NBUF, NCHUNK - 2, step=NBUF)
    def _chunks(c0):
        for cp in range(NBUF):
            c = c0 + cp
            nb = (cp + 2) % NBUF
            _wait_stores(out_hbm, rows_v, nb, t0, c - 1, ssems)
            _issue_gather(tok_hbm, idx_v, rows_v, nb, c + 2, gsems)
            _issue_pos(pos_hbm, pos_v, nb, t0, c + 2, psems)
            P = cp
            _wait_gather(tok_hbm, idx_v, rows_v, P, c, gsems)
            _wait_pos(pos_hbm, pos_v, P, t0, c, psems)
            _compute(rows_v, pos_v, P)
            _issue_stores(out_hbm, rows_v, P, t0, c, ssems)

    for c in (NCHUNK - 2, NCHUNK - 1):
        tail(c)
    for c in (NCHUNK - 3, NCHUNK - 2, NCHUNK - 1):
        _wait_stores(out_hbm, rows_v, c % NBUF, t0, c, ssems)


@jax.jit
def kernel(x, token_emb, pos_emb):
    xs = (x.astype(jnp.int32)
          .reshape(B, NW, NCHUNK, W)
          .transpose(1, 2, 0, 3)
          .reshape(ROWS))
    mesh = plsc.VectorSubcoreMesh(core_axis_name="c", subcore_axis_name="s")
    run = functools.partial(
        pl.kernel,
        out_type=jax.ShapeDtypeStruct((ROWS, D), jnp.float32),
        mesh=mesh,
        scratch_types=[
            pltpu.VMEM((B * TPW,), jnp.int32),
            pltpu.VMEM((NBUF, CR, D), jnp.float32),
            pltpu.VMEM((NBUF, W, D), jnp.float32),
            pltpu.SemaphoreType.DMA,
            pltpu.SemaphoreType.DMA,
            pltpu.SemaphoreType.DMA,
            pltpu.SemaphoreType.DMA,
            pltpu.SemaphoreType.DMA,
            pltpu.SemaphoreType.DMA,
            pltpu.SemaphoreType.DMA,
            pltpu.SemaphoreType.DMA,
            pltpu.SemaphoreType.DMA,
        ],
    )(_emb_kernel)
    out = run(token_emb, xs, pos_emb)
    return out.reshape(B, T, D)
